# Initial kernel scaffold; baseline (speedup 1.0000x reference)
import jax
import jax.numpy as jnp
from jax.experimental import pallas as pl


def kernel(h, target, neg, W):
    raise NotImplementedError("write your pallas kernel here")


# speedup vs baseline: 1.3492x; 1.3492x over previous
import functools

import jax
import jax.numpy as jnp
from jax import lax
from jax.experimental import pallas as pl
from jax.experimental.pallas import tpu as pltpu
from jax.experimental.pallas import tpu_sc as plsc

N = 8192
D = 128
V = 100000
K = 64

L = 16
NC = 2
NS = 16
NW = NC * NS
TPW = N // NW
TB = 8
NBLK = TPW // TB
RB = TB * K
GCH = 128
NG = RB // GCH


def _score_body(W_hbm, h_hbm, tgt_hbm, negf_hbm, pos_hbm, negs_hbm,
                h_v, tgt_v, negi_v, wpos_v, wneg_v, poss_v, negss_v, sem):
    wid = lax.axis_index("s") * NC + lax.axis_index("c")
    iota = lax.iota(jnp.int32, L)

    def blk_body(blk, carry):
        tok0 = wid * TPW + blk * TB
        nbase = tok0 * K
        pltpu.sync_copy(h_hbm.at[pl.ds(tok0, TB), :], h_v)
        pltpu.sync_copy(tgt_hbm.at[pl.ds(tok0, TB)], tgt_v)
        pltpu.sync_copy(negf_hbm.at[pl.ds(nbase, RB)], negi_v)
        pltpu.async_copy(W_hbm.at[tgt_v], wpos_v, sem).wait()
        for g in range(NG):
            pltpu.async_copy(
                W_hbm.at[negi_v.at[pl.ds(g * GCH, GCH)]],
                wneg_v.at[pl.ds(g * GCH, GCH), :], sem).wait()

        def tok_body(t, carry2):
            rowb = t * K
            rows = [rowb + g * L + iota for g in range(K // L)]
            accs = [jnp.zeros((L,), jnp.float32) for _ in range(K // L)]
            for j in range(D // L):
                hc = h_v[t, pl.ds(j * L, L)]
                for l in range(L):
                    d = j * L + l
                    hv = jnp.full((L,), hc[l], jnp.float32)
                    dv = jnp.full((L,), d, jnp.int32)
                    for g in range(K // L):
                        wv = plsc.load_gather(wneg_v, [rows[g], dv])
                        accs[g] = accs[g] + hv * wv
            for g in range(K // L):
                negss_v[t, pl.ds(g * L, L)] = accs[g]
            return carry2

        lax.fori_loop(0, TB, tok_body, 0)
        tok_ids = jnp.minimum(iota, TB - 1)
        pacc = jnp.zeros((L,), jnp.float32)
        for d in range(D):
            dv = jnp.full((L,), d, jnp.int32)
            ph = plsc.load_gather(h_v, [tok_ids, dv])
            pw = plsc.load_gather(wpos_v, [tok_ids, dv])
            pacc = pacc + ph * pw
        poss_v[...] = pacc
        pltpu.sync_copy(poss_v.at[pl.ds(0, TB)], pos_hbm.at[pl.ds(tok0, TB)])
        pltpu.sync_copy(negss_v, negs_hbm.at[pl.ds(tok0, TB), :])
        return carry

    lax.fori_loop(0, NBLK, blk_body, 0)


_score_call = functools.partial(
    pl.kernel,
    out_type=[
        jax.ShapeDtypeStruct((N,), jnp.float32),
        jax.ShapeDtypeStruct((N, K), jnp.float32),
    ],
    mesh=plsc.VectorSubcoreMesh(core_axis_name="c", subcore_axis_name="s"),
    compiler_params=pltpu.CompilerParams(needs_layout_passes=False),
    scratch_types=[
        pltpu.VMEM((TB, D), jnp.float32),
        pltpu.VMEM((TB,), jnp.int32),
        pltpu.VMEM((RB,), jnp.int32),
        pltpu.VMEM((TB, D), jnp.float32),
        pltpu.VMEM((RB, D), jnp.float32),
        pltpu.VMEM((L,), jnp.float32),
        pltpu.VMEM((TB, K), jnp.float32),
        pltpu.SemaphoreType.DMA,
    ],
)(_score_body)


def _loss_body(pos_ref, neg_ref, out_ref):
    p = pos_ref[...]
    z = neg_ref[...]
    sp_p = jnp.maximum(-p, 0.0) + jnp.log(1.0 + jnp.exp(-jnp.abs(p)))
    sp_n = jnp.maximum(z, 0.0) + jnp.log(1.0 + jnp.exp(-jnp.abs(z)))
    total = jnp.sum(sp_p) / N + jnp.sum(sp_n) / (N * K)
    out_ref[...] = jnp.full((1, 1), total, jnp.float32)


def kernel(h, target, neg, W):
    negf = neg.reshape(N * K).astype(jnp.int32)
    tgt = target.astype(jnp.int32)
    pos_s, neg_s = _score_call(W, h, tgt, negf)
    loss = pl.pallas_call(
        _loss_body,
        out_shape=jax.ShapeDtypeStruct((1, 1), jnp.float32),
    )(pos_s.reshape(N // D, D), neg_s.reshape(N * K // D, D))
    return loss[0, 0]


# speedup vs baseline: 5.9118x; 4.3816x over previous
import functools

import jax
import jax.numpy as jnp
from jax import lax
from jax.experimental import pallas as pl
from jax.experimental.pallas import tpu as pltpu
from jax.experimental.pallas import tpu_sc as plsc

N = 8192
D = 128
V = 100000
K = 64

L = 16
NC = 2
NS = 16
NW = NC * NS
TPW = N // NW
TB = 8
NBLK = TPW // TB
RB = TB * K
GCH = 128
NG = RB // GCH


def _score_body(W_hbm, h_hbm, tgt_hbm, negf_hbm, pos_hbm, negs_hbm,
                h_v, tgt_v, negi_v, wpos_v, wneg_v, poss_v, negss_v, sem):
    wid = lax.axis_index("s") * NC + lax.axis_index("c")
    iota = lax.iota(jnp.int32, L)
    perms = [(iota + i) & (L - 1) for i in range(L)]

    def blk_body(blk, carry):
        tok0 = wid * TPW + blk * TB
        nbase = tok0 * K
        pltpu.sync_copy(h_hbm.at[pl.ds(tok0, TB), :], h_v)
        pltpu.sync_copy(tgt_hbm.at[pl.ds(tok0, TB)], tgt_v)
        pltpu.sync_copy(negf_hbm.at[pl.ds(nbase, RB)], negi_v)
        descs = [pltpu.async_copy(W_hbm.at[tgt_v], wpos_v, sem)]
        for g in range(NG):
            descs.append(pltpu.async_copy(
                W_hbm.at[negi_v.at[pl.ds(g * GCH, GCH)]],
                wneg_v.at[pl.ds(g * GCH, GCH), :], sem))
        for dsc in descs:
            dsc.wait()

        def tok_body(t, carry2):
            rowb = t * K
            rows = [rowb + g * L + iota for g in range(K // L)]
            tsp = jnp.full((L,), t, jnp.int32)
            accs = [jnp.zeros((L,), jnp.float32) for _ in range(K // L)]
            for j in range(D // L):
                for i in range(L):
                    dv = perms[i] + (j * L)
                    hv = plsc.load_gather(h_v, [tsp, dv])
                    for g in range(K // L):
                        wv = plsc.load_gather(wneg_v, [rows[g], dv])
                        accs[g] = accs[g] + hv * wv
            for g in range(K // L):
                negss_v[t, pl.ds(g * L, L)] = accs[g]
            return carry2

        lax.fori_loop(0, TB, tok_body, 0)
        tok_ids = jnp.minimum(iota, TB - 1)
        pacc = jnp.zeros((L,), jnp.float32)
        for j in range(D // L):
            for i in range(L):
                dv = perms[i] + (j * L)
                ph = plsc.load_gather(h_v, [tok_ids, dv])
                pw = plsc.load_gather(wpos_v, [tok_ids, dv])
                pacc = pacc + ph * pw
        poss_v[...] = pacc
        pltpu.sync_copy(poss_v.at[pl.ds(0, TB)], pos_hbm.at[pl.ds(tok0, TB)])
        pltpu.sync_copy(negss_v, negs_hbm.at[pl.ds(tok0, TB), :])
        return carry

    lax.fori_loop(0, NBLK, blk_body, 0)


_score_call = functools.partial(
    pl.kernel,
    out_type=[
        jax.ShapeDtypeStruct((N,), jnp.float32),
        jax.ShapeDtypeStruct((N, K), jnp.float32),
    ],
    mesh=plsc.VectorSubcoreMesh(core_axis_name="c", subcore_axis_name="s"),
    compiler_params=pltpu.CompilerParams(needs_layout_passes=False),
    scratch_types=[
        pltpu.VMEM((TB, D), jnp.float32),
        pltpu.VMEM((TB,), jnp.int32),
        pltpu.VMEM((RB,), jnp.int32),
        pltpu.VMEM((TB, D), jnp.float32),
        pltpu.VMEM((RB, D), jnp.float32),
        pltpu.VMEM((L,), jnp.float32),
        pltpu.VMEM((TB, K), jnp.float32),
        pltpu.SemaphoreType.DMA,
    ],
)(_score_body)


def _loss_body(pos_ref, neg_ref, out_ref):
    p = pos_ref[...]
    z = neg_ref[...]
    sp_p = jnp.maximum(-p, 0.0) + jnp.log(1.0 + jnp.exp(-jnp.abs(p)))
    sp_n = jnp.maximum(z, 0.0) + jnp.log(1.0 + jnp.exp(-jnp.abs(z)))
    total = jnp.sum(sp_p) / N + jnp.sum(sp_n) / (N * K)
    out_ref[...] = jnp.full((1, 1), total, jnp.float32)


def kernel(h, target, neg, W):
    negf = neg.reshape(N * K).astype(jnp.int32)
    tgt = target.astype(jnp.int32)
    pos_s, neg_s = _score_call(W, h, tgt, negf)
    loss = pl.pallas_call(
        _loss_body,
        out_shape=jax.ShapeDtypeStruct((1, 1), jnp.float32),
    )(pos_s.reshape(N // D, D), neg_s.reshape(N * K // D, D))
    return loss[0, 0]


# speedup vs baseline: 6.4709x; 1.0946x over previous
import functools

import jax
import jax.numpy as jnp
from jax import lax
from jax.experimental import pallas as pl
from jax.experimental.pallas import tpu as pltpu
from jax.experimental.pallas import tpu_sc as plsc

N = 8192
D = 128
V = 100000
K = 64

L = 16
NC = 2
NS = 16
NW = NC * NS
TPW = N // NW
CH = 128
TPC = CH // K
NCHUNK = TPW * K // CH
NIT = NCHUNK // 2
PQ = 64
NG = K // L
SB = 64


def _score_body(W_hbm, h_hbm, tgt_hbm, negf_hbm, pos_hbm, negs_hbm,
                h_v, tgt_v, negi_v, wch0, wch1, wpos0, wpos1,
                poss_v, negss_v, semA, semB, semP0, semP1):
    wid = lax.axis_index("s") * NC + lax.axis_index("c")
    base = wid * TPW
    iota = lax.iota(jnp.int32, L)

    pltpu.sync_copy(h_hbm.at[pl.ds(base, TPW), :], h_v)
    pltpu.sync_copy(tgt_hbm.at[pl.ds(base, TPW)], tgt_v)
    pltpu.sync_copy(negf_hbm.at[pl.ds(base * K, TPW * K)], negi_v)
    pltpu.async_copy(W_hbm.at[negi_v.at[pl.ds(0, CH)]], wch0, semA)
    pltpu.async_copy(W_hbm.at[negi_v.at[pl.ds(CH, CH)]], wch1, semB)
    pltpu.async_copy(W_hbm.at[tgt_v.at[pl.ds(0, PQ)]], wpos0, semP0)
    pltpu.async_copy(W_hbm.at[tgt_v.at[pl.ds(PQ, PQ)]], wpos1, semP1)

    def compute_token(tw, buf, s2):
        tsp = jnp.full((L,), tw, jnp.int32)
        rows = [s2 * K + g * L + iota for g in range(NG)]

        def jbody(j, accs):
            j16 = j * L
            new = list(accs)
            for ii in range(L):
                dv = ((iota + ii) & (L - 1)) + j16
                hv = plsc.load_gather(h_v, [tsp, dv])
                for g in range(NG):
                    wv = plsc.load_gather(buf, [rows[g], dv])
                    new[g] = new[g] + hv * wv
            return tuple(new)

        accs = lax.fori_loop(
            0, D // L, jbody,
            tuple(jnp.zeros((L,), jnp.float32) for _ in range(NG)))
        lt = tw & (SB - 1)
        for g in range(NG):
            negss_v[lt, pl.ds(g * L, L)] = accs[g]

    def it_body(i, carry):
        for s, (buf, sem_) in enumerate(((wch0, semA), (wch1, semB))):
            c = 2 * i + s
            pltpu.make_async_copy(
                W_hbm.at[negi_v.at[pl.ds(0, CH)]], buf, sem_).wait()
            for s2 in range(TPC):
                compute_token(c * TPC + s2, buf, s2)

            @pl.when(i < NIT - 1)
            def _fire():
                pltpu.async_copy(
                    W_hbm.at[negi_v.at[pl.ds((c + 2) * CH, CH)]], buf, sem_)

        @pl.when((i & (SB // 4 - 1)) == SB // 4 - 1)
        def _flush():
            off = base + (i // (SB // 4)) * SB
            pltpu.sync_copy(negss_v, negs_hbm.at[pl.ds(off, SB), :])

        return carry

    lax.fori_loop(0, NIT, it_body, 0)

    for q in range(TPW // PQ):
        wpos, semP = ((wpos0, semP0), (wpos1, semP1))[q & 1]
        pltpu.make_async_copy(
            W_hbm.at[tgt_v.at[pl.ds(0, PQ)]], wpos, semP).wait()

        def tg_body(tg, carry):
            tok_ids = q * PQ + tg * L + iota
            lrows = tg * L + iota

            def pj(j, pacc):
                j16 = j * L
                acc = pacc
                for ii in range(L):
                    dv = ((iota + ii) & (L - 1)) + j16
                    ph = plsc.load_gather(h_v, [tok_ids, dv])
                    pw = plsc.load_gather(wpos, [lrows, dv])
                    acc = acc + ph * pw
                return acc

            pacc = lax.fori_loop(0, D // L, pj, jnp.zeros((L,), jnp.float32))
            poss_v[pl.ds(tg * L, L)] = pacc
            return carry

        lax.fori_loop(0, PQ // L, tg_body, 0)
        pltpu.sync_copy(poss_v.at[pl.ds(0, PQ)],
                        pos_hbm.at[pl.ds(base + q * PQ, PQ)])
        if q + 2 < TPW // PQ:
            pltpu.async_copy(
                W_hbm.at[tgt_v.at[pl.ds((q + 2) * PQ, PQ)]], wpos, semP)


_score_call = functools.partial(
    pl.kernel,
    out_type=[
        jax.ShapeDtypeStruct((N,), jnp.float32),
        jax.ShapeDtypeStruct((N, K), jnp.float32),
    ],
    mesh=plsc.VectorSubcoreMesh(core_axis_name="c", subcore_axis_name="s"),
    compiler_params=pltpu.CompilerParams(needs_layout_passes=False),
    scratch_types=[
        pltpu.VMEM((TPW, D), jnp.float32),
        pltpu.VMEM((TPW,), jnp.int32),
        pltpu.VMEM((TPW * K,), jnp.int32),
        pltpu.VMEM((CH, D), jnp.float32),
        pltpu.VMEM((CH, D), jnp.float32),
        pltpu.VMEM((PQ, D), jnp.float32),
        pltpu.VMEM((PQ, D), jnp.float32),
        pltpu.VMEM((PQ,), jnp.float32),
        pltpu.VMEM((SB, K), jnp.float32),
        pltpu.SemaphoreType.DMA,
        pltpu.SemaphoreType.DMA,
        pltpu.SemaphoreType.DMA,
        pltpu.SemaphoreType.DMA,
    ],
)(_score_body)


def _loss_body(pos_ref, neg_ref, out_ref):
    p = pos_ref[...]
    z = neg_ref[...]
    sp_p = jnp.maximum(-p, 0.0) + jnp.log(1.0 + jnp.exp(-jnp.abs(p)))
    sp_n = jnp.maximum(z, 0.0) + jnp.log(1.0 + jnp.exp(-jnp.abs(z)))
    total = jnp.sum(sp_p) / N + jnp.sum(sp_n) / (N * K)
    out_ref[...] = jnp.full((1, 1), total, jnp.float32)


def kernel(h, target, neg, W):
    negf = neg.reshape(N * K).astype(jnp.int32)
    tgt = target.astype(jnp.int32)
    pos_s, neg_s = _score_call(W, h, tgt, negf)
    loss = pl.pallas_call(
        _loss_body,
        out_shape=jax.ShapeDtypeStruct((1, 1), jnp.float32),
    )(pos_s.reshape(N // D, D), neg_s.reshape(N * K // D, D))
    return loss[0, 0]
